# Initial kernel scaffold; baseline (speedup 1.0000x reference)
import jax
import jax.numpy as jnp
from jax.experimental import pallas as pl


def kernel(x, edge_index, W1, a1_src, a1_dst, b1, W2, a2_src, a2_dst, b2, W3, a3_src, a3_dst, b3, W4, a4_src, a4_dst, b4):
    raise NotImplementedError("write your pallas kernel here")


# speedup vs baseline: 5.2665x; 5.2665x over previous
import dataclasses
import functools

import jax
import jax.numpy as jnp
from jax import lax
from jax.experimental import pallas as pl
from jax.experimental.pallas import tpu as pltpu
from jax.experimental.pallas import tpu_sc as plsc

N = 10000
NPAD = 10240
E = 160000
EN = E + N
EPAD = 172032
NSCAN = EPAD // 2048
RPT = NPAD // 16
CAPT = 88 * 128
JROW = RPT
BN = 640
GN = NPAD // BN

_MESH = plsc.VectorSubcoreMesh(core_axis_name="c", subcore_axis_name="s")

_SC_PARAMS = pltpu.CompilerParams()
if "needs_layout_passes" in pltpu.CompilerParams.__dataclass_fields__:
    _SC_PARAMS = dataclasses.replace(_SC_PARAMS, needs_layout_passes=False)


def _mm_body(apply_act, in3d, cin, cout,
             x_ref, w_ref, a8_ref, b_ref, hc_ref, asadt_ref, m_ref,
             cmax_ref):
    n = pl.program_id(0)
    j = pl.program_id(1)
    k = pl.program_id(2)
    xb = x_ref[0] if in3d else x_ref[...]
    if apply_act:
        xb = xb + b_ref[0, 0]
        xb = jnp.where(xb >= 0.0, xb, 0.01 * xb)
    wb = w_ref[...]
    part = jnp.dot(xb, wb, preferred_element_type=jnp.float32)

    @pl.when(k == 0)
    def _():
        hc_ref[0] = part

    @pl.when(k > 0)
    def _():
        hc_ref[0] = hc_ref[0] + part

    wa = jnp.dot(wb, a8_ref[...], preferred_element_type=jnp.float32)
    contrib_t = lax.dot_general(wa, xb, (((0,), (1,)), ((), ())),
                                preferred_element_type=jnp.float32)
    first = jnp.logical_and(j == 0, k == 0)

    @pl.when(first)
    def _():
        asadt_ref[...] = contrib_t

    @pl.when(jnp.logical_not(first))
    def _():
        asadt_ref[...] = asadt_ref[...] + contrib_t

    @pl.when(jnp.logical_and(j == cout - 1, k == cin - 1))
    def _():
        av = asadt_ref[...]
        mas = jnp.max(av[0])
        mad = jnp.max(av[1])

        @pl.when(n == 0)
        def _():
            cmax_ref[0] = mas
            cmax_ref[1] = mad

        @pl.when(n > 0)
        def _():
            cmax_ref[0] = jnp.maximum(cmax_ref[0], mas)
            cmax_ref[1] = jnp.maximum(cmax_ref[1], mad)

        m_ref[...] = jnp.full((1, 128), cmax_ref[0] + cmax_ref[1],
                              jnp.float32)


def _tc_mm(din, dout, apply_act, in3d):
    cin = din // 128
    cout = dout // 128
    if in3d:
        x_spec = pl.BlockSpec((1, BN, 128), lambda n, j, k: (k, n, 0))
    else:
        x_spec = pl.BlockSpec((BN, 128), lambda n, j, k: (n, k))
    return pl.pallas_call(
        functools.partial(_mm_body, apply_act, in3d, cin, cout),
        grid=(GN, cout, cin),
        in_specs=[
            x_spec,
            pl.BlockSpec((128, 128), lambda n, j, k: (k, j)),
            pl.BlockSpec((128, 8), lambda n, j, k: (j, 0)),
            pl.BlockSpec((1, 1, 128), lambda n, j, k: (k, 0, 0)),
        ],
        out_specs=[
            pl.BlockSpec((1, BN, 128), lambda n, j, k: (j, n, 0)),
            pl.BlockSpec((8, BN), lambda n, j, k: (0, n)),
            pl.BlockSpec((1, 128), lambda n, j, k: (0, 0)),
        ],
        out_shape=[
            jax.ShapeDtypeStruct((cout, NPAD, 128), jnp.float32),
            jax.ShapeDtypeStruct((8, NPAD), jnp.float32),
            jax.ShapeDtypeStruct((1, 128), jnp.float32),
        ],
        scratch_shapes=[pltpu.SMEM((2,), jnp.float32)],
    )


def _agg_body(nchunks, cc_per_sc,
              hc_hbm, sd_hbm, asadt_hbm, m_hbm, out_hbm,
              as_v, ad_v, m_v, sd_v, srcc, dstc, sloc, pb_v,
              rows0, rows1, acc_v, sem0, sem1):
    cid = lax.axis_index("c")
    tid = lax.axis_index("s")
    lo = tid * RPT
    iota16 = lax.iota(jnp.int32, 16)
    zf16 = jnp.zeros((16,), jnp.float32)

    pltpu.sync_copy(asadt_hbm.at[0], as_v)
    pltpu.sync_copy(asadt_hbm.at[1, pl.ds(lo, RPT)], ad_v.at[pl.ds(0, RPT)])
    ad_v[pl.ds(RPT, 16)] = zf16
    pltpu.sync_copy(m_hbm.at[pl.ds(0, 16)], m_v)
    mv = m_v[...]

    def scan_outer(bi, cnt):
        pltpu.sync_copy(sd_hbm.at[:, pl.ds(bi * 2048, 2048)], sd_v)

        def scan_inner(ii, cnt):
            sv = sd_v[0, pl.ds(ii * 16, 16)]
            dv = sd_v[1, pl.ds(ii * 16, 16)]
            ev = jnp.full((16,), bi * 2048 + ii * 16, jnp.int32) + iota16
            msk = ((dv >= lo) & (dv < lo + RPT) & (ev < EN))
            off = jnp.minimum(cnt, CAPT - 32)
            plsc.store_compressed(srcc.at[pl.ds(off, 16)], sv, mask=msk)
            plsc.store_compressed(dstc.at[pl.ds(off, 16)], dv - lo,
                                  mask=msk)
            return cnt + jnp.sum(msk.astype(jnp.int32))

        return lax.fori_loop(0, 128, scan_inner, cnt)

    cnt = lax.fori_loop(0, NSCAN, scan_outer, jnp.int32(0))

    offj = jnp.minimum(cnt, CAPT - 32)
    srcc[pl.ds(offj, 16)] = jnp.zeros((16,), jnp.int32)
    srcc[pl.ds(offj + 16, 16)] = jnp.zeros((16,), jnp.int32)
    dstc[pl.ds(offj, 16)] = jnp.full((16,), JROW, jnp.int32)
    dstc[pl.ds(offj + 16, 16)] = jnp.full((16,), JROW, jnp.int32)
    ngroups2 = jnp.maximum((cnt + 31) // 32 * 2, 2)

    def chunk_body(cc, carry):
        chunk = cid * cc_per_sc + cc
        rowbase = chunk * NPAD

        @pl.loop(0, RPT + 8)
        def _(r):
            for c in range(8):
                plsc.store_scatter(acc_v,
                                   [jnp.full((16,), r, jnp.int32),
                                    iota16 + c * 16], zf16)

        @pl.loop(0, RPT + 16, step=16)
        def _(i):
            sloc[pl.ds(i, 16)] = zf16

        def idxv(g):
            return srcc[pl.ds(g * 16, 16)] + rowbase

        pltpu.async_copy(hc_hbm.at[idxv(0)], rows0, sem0)
        pltpu.async_copy(hc_hbm.at[idxv(1)], rows1, sem1)

        def pair_body(pp, carry):
            for b, rows_b, sem_b in ((0, rows0, sem0), (1, rows1, sem1)):
                g = pp * 2 + b
                pltpu.make_async_copy(hc_hbm.at[idxv(g)], rows_b,
                                      sem_b).wait()
                s16 = srcc[pl.ds(g * 16, 16)]
                dl16 = dstc[pl.ds(g * 16, 16)]
                av = plsc.load_gather(as_v, [s16])
                bv = plsc.load_gather(ad_v, [dl16])
                lg = av + bv
                lg = jnp.where(lg >= 0.0, lg, 0.2 * lg)
                p16 = jnp.exp(lg - mv)
                pb_v[...] = p16
                for l in range(16):
                    plsc.addupdate_scatter(sloc, [dl16], p16,
                                           mask=iota16 == l)
                for l in range(16):
                    lsp = jnp.full((16,), l, jnp.int32)
                    psp = plsc.load_gather(pb_v, [lsp])
                    dsp = plsc.load_gather(dstc,
                                           [jnp.full((16,), g * 16 + l,
                                                     jnp.int32)])
                    for c in range(8):
                        plsc.addupdate_scatter(
                            acc_v, [dsp, iota16 + c * 16],
                            rows_b[l, pl.ds(c * 16, 16)] * psp)
                gn = jnp.minimum(g + 2, ngroups2 - 1)
                pltpu.async_copy(hc_hbm.at[idxv(gn)], rows_b, sem_b)
            return carry

        lax.fori_loop(0, ngroups2 // 2, pair_body, 0)
        pltpu.make_async_copy(hc_hbm.at[idxv(0)], rows0, sem0).wait()
        pltpu.make_async_copy(hc_hbm.at[idxv(0)], rows1, sem1).wait()

        @pl.loop(0, RPT)
        def _(r):
            rsp = jnp.full((16,), r, jnp.int32)
            ssp = plsc.load_gather(sloc, [rsp])
            inv = 1.0 / jnp.maximum(ssp, 1e-30)
            for c in range(8):
                cols = iota16 + c * 16
                v = plsc.load_gather(acc_v, [rsp, cols])
                plsc.store_scatter(acc_v, [rsp, cols], v * inv)

        pltpu.sync_copy(acc_v.at[pl.ds(0, RPT)],
                        out_hbm.at[pl.ds(rowbase + lo, RPT)])
        return carry

    lax.fori_loop(0, cc_per_sc, chunk_body, 0)


def _sc_agg(nchunks):
    cc_per_sc = nchunks // 2
    return pl.kernel(
        functools.partial(_agg_body, nchunks, cc_per_sc),
        out_type=jax.ShapeDtypeStruct((nchunks * NPAD, 128), jnp.float32),
        mesh=_MESH,
        scratch_types=[
            pltpu.VMEM((NPAD,), jnp.float32),
            pltpu.VMEM((RPT + 16,), jnp.float32),
            pltpu.VMEM((16,), jnp.float32),
            pltpu.VMEM((2, 2048), jnp.int32),
            pltpu.VMEM((CAPT,), jnp.int32),
            pltpu.VMEM((CAPT,), jnp.int32),
            pltpu.VMEM((RPT + 16,), jnp.float32),
            pltpu.VMEM((16,), jnp.float32),
            pltpu.VMEM((16, 128), jnp.float32),
            pltpu.VMEM((16, 128), jnp.float32),
            pltpu.VMEM((RPT + 8, 128), jnp.float32),
            pltpu.SemaphoreType.DMA,
            pltpu.SemaphoreType.DMA,
        ],
        compiler_params=_SC_PARAMS,
    )


def _asm_body(pre_ref, b_ref, o_ref):
    o_ref[...] = pre_ref[0] + b_ref[0, 0]


_assemble = pl.pallas_call(
    _asm_body,
    grid=(25, 2),
    in_specs=[
        pl.BlockSpec((1, 400, 128), lambda n, c: (c, n, 0)),
        pl.BlockSpec((1, 1, 128), lambda n, c: (c, 0, 0)),
    ],
    out_specs=pl.BlockSpec((400, 128), lambda n, c: (n, c)),
    out_shape=jax.ShapeDtypeStruct((N, 256), jnp.float32),
)


def kernel(x, edge_index, W1, a1_src, a1_dst, b1, W2, a2_src, a2_dst, b2,
           W3, a3_src, a3_dst, b3, W4, a4_src, a4_dst, b4):
    loop = jnp.arange(N, dtype=jnp.int32)
    sd = jnp.concatenate(
        [edge_index, jnp.stack([loop, loop]),
         jnp.zeros((2, EPAD - EN), jnp.int32)], axis=1)

    layers = [(W1, a1_src, a1_dst), (W2, a2_src, a2_dst),
              (W3, a3_src, a3_dst), (W4, a4_src, a4_dst)]
    biases = [b1, b2, b3, b4]

    xcur = jnp.pad(x, ((0, NPAD - N), (0, 0)))
    for li, (W, a_s, a_d) in enumerate(layers):
        din, dout = W.shape
        nchunks = dout // 128
        a8 = jnp.zeros((dout, 8), jnp.float32)
        a8 = a8.at[:, 0].set(a_s).at[:, 1].set(a_d)
        bprev = biases[li - 1] if li > 0 else jnp.zeros((din,), jnp.float32)
        b3d = bprev.reshape(din // 128, 1, 128)
        hc, asadt, m = _tc_mm(din, dout, li > 0, li > 0)(xcur, W, a8, b3d)
        outf = _sc_agg(nchunks)(hc.reshape(nchunks * NPAD, 128), sd, asadt,
                                m.reshape(128))
        xcur = outf.reshape(nchunks, NPAD, 128)

    return _assemble(xcur, b4.reshape(2, 1, 128))
